# baseline (speedup 1.0000x reference)
import jax
import jax.numpy as jnp
from jax import lax
from jax.experimental import pallas as pl
from jax.experimental.pallas import tpu as pltpu
from jax.experimental.pallas import tpu_sc as plsc

N = 10000
E = 160000
R = 8
D = 128

NC = 2
NS = 16
NW = NC * NS
CHUNK = 128
NCHUNK = 40
EPW = CHUNK * NCHUNK
E_PAD = EPW * NW
N_PAD = 10240
RPT = N_PAD // NS
ND8 = N_PAD // 8
RPT8 = ND8 // NS
BN = 400
BN8 = BN // 8

_MESH = plsc.VectorSubcoreMesh(
    core_axis_name="c", subcore_axis_name="s", num_cores=NC, num_subcores=NS
)


def _dense_body(x_ref, wrel_ref, wself_ref, b_ref, t_ref, s_ref):
    x = x_ref[...]
    s_ref[...] = (
        lax.dot_general(x, wself_ref[...], (((1,), (1,)), ((), ())),
                        preferred_element_type=jnp.float32)
        + b_ref[...]
    )
    for r in range(R):
        t_ref[r] = lax.dot_general(x, wrel_ref[r], (((1,), (0,)), ((), ())),
                                   preferred_element_type=jnp.float32)


def _dense(x, wrel, wself, b):
    return pl.pallas_call(
        _dense_body,
        grid=(N // BN,),
        in_specs=[
            pl.BlockSpec((BN, D), lambda i: (i, 0)),
            pl.BlockSpec((R, D, D), lambda i: (0, 0, 0)),
            pl.BlockSpec((D, D), lambda i: (0, 0)),
            pl.BlockSpec((1, D), lambda i: (0, 0)),
        ],
        out_specs=[
            pl.BlockSpec((R, BN, D), lambda i: (0, i, 0)),
            pl.BlockSpec((BN, D), lambda i: (i, 0)),
        ],
        out_shape=[
            jax.ShapeDtypeStruct((R, N, D), jnp.float32),
            jax.ShapeDtypeStruct((N, D), jnp.float32),
        ],
    )(x, wrel, wself, b.reshape(1, D))


def _gidx_body(src_ref, typ_ref, dst_ref, g_ref, dg_ref, ds_ref):
    g_ref[...] = typ_ref[...] * N + src_ref[...]
    dg_ref[...] = lax.bitwise_and(dst_ref[...], 7)
    ds_ref[...] = lax.shift_right_logical(dst_ref[...], 3)


def _gidx(src, typ, dst):
    return pl.pallas_call(
        _gidx_body,
        out_shape=[
            jax.ShapeDtypeStruct((NW, NCHUNK, CHUNK), jnp.int32),
            jax.ShapeDtypeStruct((NW, NCHUNK, CHUNK), jnp.int32),
            jax.ShapeDtypeStruct((NW, NCHUNK, CHUNK), jnp.int32),
        ],
    )(src, typ, dst)


K = 2
NGROUP = NCHUNK // K


def _sc_pipe(idx2, dst2, table_h, acc_sh, rows, gsem, ssem, dummy_h):
    def group(g, c):
        @pl.when(g > 0)
        def _():
            for b in range(K):
                pltpu.make_async_copy(dummy_h, rows[b], ssem).wait()

        gd = []
        for b in range(K):
            gd.append(pltpu.async_copy(
                table_h.at[idx2.at[g * K + b]], rows[b], gsem))
        for b in range(K):
            gd[b].wait()
        for b in range(K):
            pltpu.async_copy(rows[b], acc_sh.at[dst2.at[g * K + b]], ssem,
                             add=True)
        return c

    lax.fori_loop(0, NGROUP, group, 0)
    for b in range(K):
        pltpu.make_async_copy(dummy_h, rows[b], ssem).wait()


def _sc_body(gidx_h, dst_h, table_h, zr_h,
             pout_h,
             agg_sh, gia, dsa, gib, dsb, r0, r1,
             gsem, ssem, isem):
    cid = lax.axis_index("c")
    sid = lax.axis_index("s")
    wid = cid * NS + sid
    row0 = sid * RPT
    rows = (r0, r1)
    dummy_r = zr_h.at[pl.ds(0, CHUNK)]
    pltpu.sync_copy(zr_h, agg_sh.at[pl.ds(row0, RPT)])
    pltpu.sync_copy(gidx_h.at[wid, 0], gia)
    pltpu.sync_copy(dst_h.at[wid, 0], dsa)
    plsc.subcore_barrier()

    def half(gi_cur, ds_cur, gi_nxt, ds_nxt, g):
        @pl.when(g > 0)
        def _():
            for b in range(K):
                pltpu.make_async_copy(dummy_r, rows[b], ssem).wait()

        @pl.when(g < NGROUP - 1)
        def _():
            pltpu.async_copy(gidx_h.at[wid, g + 1], gi_nxt, isem)
            pltpu.async_copy(dst_h.at[wid, g + 1], ds_nxt, isem)

        gd = []
        for b in range(K):
            gd.append(pltpu.async_copy(table_h.at[gi_cur.at[b]], rows[b],
                                       gsem))
        for b in range(K):
            gd[b].wait()
        for b in range(K):
            pltpu.async_copy(rows[b], agg_sh.at[ds_cur.at[b]], ssem, add=True)
        @pl.when(g < NGROUP - 1)
        def _():
            pltpu.make_async_copy(gidx_h.at[wid, 0], gi_nxt, isem).wait()
            pltpu.make_async_copy(dst_h.at[wid, 0], ds_nxt, isem).wait()

    def pair(i, c):
        half(gia, dsa, gib, dsb, 2 * i)
        half(gib, dsb, gia, dsa, 2 * i + 1)
        return c

    lax.fori_loop(0, NGROUP // 2, pair, 0)
    for b in range(K):
        pltpu.make_async_copy(dummy_r, rows[b], ssem).wait()
    plsc.subcore_barrier()
    pltpu.sync_copy(agg_sh.at[pl.ds(row0, RPT)],
                    pout_h.at[cid, pl.ds(row0, RPT)])


_sc_agg = pl.kernel(
    _sc_body,
    out_type=jax.ShapeDtypeStruct((NC, N_PAD, D), jnp.float32),
    mesh=_MESH,
    scratch_types=[
        pltpu.VMEM_SHARED((N_PAD, D), jnp.float32),
        pltpu.VMEM((K, CHUNK), jnp.int32),
        pltpu.VMEM((K, CHUNK), jnp.int32),
        pltpu.VMEM((K, CHUNK), jnp.int32),
        pltpu.VMEM((K, CHUNK), jnp.int32),
        pltpu.VMEM((CHUNK, D), jnp.float32),
        pltpu.VMEM((CHUNK, D), jnp.float32),
        pltpu.SemaphoreType.DMA,
        pltpu.SemaphoreType.DMA,
        pltpu.SemaphoreType.DMA,
    ],
)


def _sc_deg_body(dg_h, ds_h, onehot_h, zr_h,
                 pdeg_h,
                 deg_sh, dg2, ds2, r0, r1,
                 gsem, ssem):
    cid = lax.axis_index("c")
    sid = lax.axis_index("s")
    wid = cid * NS + sid
    row0 = sid * RPT8
    pltpu.sync_copy(zr_h.at[pl.ds(0, RPT8)], deg_sh.at[pl.ds(row0, RPT8)])
    pltpu.sync_copy(dg_h.at[wid], dg2)
    pltpu.sync_copy(ds_h.at[wid], ds2)
    plsc.subcore_barrier()
    _sc_pipe(dg2, ds2, onehot_h, deg_sh, (r0, r1), gsem, ssem,
             zr_h.at[pl.ds(0, CHUNK)])
    plsc.subcore_barrier()
    pltpu.sync_copy(deg_sh.at[pl.ds(row0, RPT8)],
                    pdeg_h.at[cid, pl.ds(row0, RPT8)])


_sc_deg = pl.kernel(
    _sc_deg_body,
    out_type=jax.ShapeDtypeStruct((NC, ND8, D), jnp.float32),
    mesh=_MESH,
    scratch_types=[
        pltpu.VMEM_SHARED((ND8, D), jnp.float32),
        pltpu.VMEM((NCHUNK, CHUNK), jnp.int32),
        pltpu.VMEM((NCHUNK, CHUNK), jnp.int32),
        pltpu.VMEM((CHUNK, D), jnp.float32),
        pltpu.VMEM((CHUNK, D), jnp.float32),
        pltpu.SemaphoreType.DMA,
        pltpu.SemaphoreType.DMA,
    ],
)


def _degsum_body(pd_ref, d_ref):
    d8 = pd_ref[0] + pd_ref[1]
    ci = lax.broadcasted_iota(jnp.int32, (D, 8), 0)
    ki = lax.broadcasted_iota(jnp.int32, (D, 8), 1)
    sel = (ci == ki * 16).astype(jnp.float32)
    d_ref[...] = lax.dot_general(d8, sel, (((1,), (0,)), ((), ())),
                                 preferred_element_type=jnp.float32)


_BD = 160


def _degsum(pdeg):
    return pl.pallas_call(
        _degsum_body,
        grid=(ND8 // _BD,),
        in_specs=[pl.BlockSpec((NC, _BD, D), lambda i: (0, i, 0))],
        out_specs=pl.BlockSpec((_BD, 8), lambda i: (i, 0)),
        out_shape=jax.ShapeDtypeStruct((ND8, 8), jnp.float32),
    )(pdeg)


def _combine_body(s_ref, p_ref, d_ref, o_ref):
    deg = jnp.maximum(d_ref[...], 1.0)
    agg = p_ref[0] + p_ref[1]
    o_ref[...] = jnp.maximum(s_ref[...] + agg / deg, 0.0)


def _combine(selfout, p, deg):
    return pl.pallas_call(
        _combine_body,
        grid=(N // BN,),
        in_specs=[
            pl.BlockSpec((BN, D), lambda i: (i, 0)),
            pl.BlockSpec((NC, BN, D), lambda i: (0, i, 0)),
            pl.BlockSpec((BN, 1), lambda i: (i, 0)),
        ],
        out_specs=pl.BlockSpec((BN, D), lambda i: (i, 0)),
        out_shape=jax.ShapeDtypeStruct((N, D), jnp.float32),
    )(selfout, p, deg)


def kernel(local_nodes, edge_index_local, edge_type_local, node_emb,
           W_rel0, Wself_w0, Wself_b0, W_rel1, Wself_w1, Wself_b1):
    x = jnp.take(node_emb, local_nodes, axis=0)
    pad = E_PAD - E
    src = jnp.concatenate(
        [edge_index_local[0], jnp.zeros((pad,), jnp.int32)]
    ).reshape(NW, NCHUNK, CHUNK)
    typ = jnp.concatenate(
        [edge_type_local, jnp.zeros((pad,), jnp.int32)]
    ).reshape(NW, NCHUNK, CHUNK)
    dst = jnp.concatenate(
        [edge_index_local[1], jnp.full((pad,), N, jnp.int32)]
    ).reshape(NW, NCHUNK, CHUNK)
    zr = jnp.zeros((RPT, D), jnp.float32)
    onehot = (
        lax.broadcasted_iota(jnp.int32, (8, D), 1)
        == lax.broadcasted_iota(jnp.int32, (8, D), 0) * 16
    ).astype(jnp.float32)

    gidx, dgidx, dsidx = _gidx(src, typ, dst)
    pdeg = _sc_deg(dgidx, dsidx, onehot, zr)
    deg = _degsum(pdeg).reshape(N_PAD, 1)
    t0, s0 = _dense(x, W_rel0, Wself_w0, Wself_b0)
    gidx4 = gidx.reshape(NW, NGROUP, K, CHUNK)
    dst4 = dst.reshape(NW, NGROUP, K, CHUNK)
    p0 = _sc_agg(gidx4, dst4, t0.reshape(R * N, D), zr)
    x1 = _combine(s0, p0, deg)
    t1, s1 = _dense(x1, W_rel1, Wself_w1, Wself_b1)
    p1 = _sc_agg(gidx4, dst4, t1.reshape(R * N, D), zr)
    x2 = _combine(s1, p1, deg)
    return x2


# speedup vs baseline: 1.0779x; 1.0779x over previous
import jax
import jax.numpy as jnp
from jax import lax
from jax.experimental import pallas as pl
from jax.experimental.pallas import tpu as pltpu
from jax.experimental.pallas import tpu_sc as plsc

N = 10000
E = 160000
R = 8
D = 128

NC = 2
NS = 16
NW = NC * NS
CHUNK = 80
NCHUNK = 64
EPW = CHUNK * NCHUNK
E_PAD = EPW * NW
N_PAD = 10240
RPT = N_PAD // NS
ND8 = N_PAD // 8
RPT8 = ND8 // NS
BN = 400
BN8 = BN // 8

_MESH = plsc.VectorSubcoreMesh(
    core_axis_name="c", subcore_axis_name="s", num_cores=NC, num_subcores=NS
)


def _dense_body(x_ref, wrel_ref, wself_ref, b_ref, t_ref, s_ref):
    x = x_ref[...]
    s_ref[...] = (
        lax.dot_general(x, wself_ref[...], (((1,), (1,)), ((), ())),
                        preferred_element_type=jnp.float32)
        + b_ref[...]
    )
    for r in range(R):
        t_ref[r] = lax.dot_general(x, wrel_ref[r], (((1,), (0,)), ((), ())),
                                   preferred_element_type=jnp.float32)


def _dense(x, wrel, wself, b):
    return pl.pallas_call(
        _dense_body,
        grid=(N // BN,),
        in_specs=[
            pl.BlockSpec((BN, D), lambda i: (i, 0)),
            pl.BlockSpec((R, D, D), lambda i: (0, 0, 0)),
            pl.BlockSpec((D, D), lambda i: (0, 0)),
            pl.BlockSpec((1, D), lambda i: (0, 0)),
        ],
        out_specs=[
            pl.BlockSpec((R, BN, D), lambda i: (0, i, 0)),
            pl.BlockSpec((BN, D), lambda i: (i, 0)),
        ],
        out_shape=[
            jax.ShapeDtypeStruct((R, N, D), jnp.float32),
            jax.ShapeDtypeStruct((N, D), jnp.float32),
        ],
    )(x, wrel, wself, b.reshape(1, D))


def _gidx_body(src_ref, typ_ref, dst_ref, g_ref, dg_ref, ds_ref):
    g_ref[...] = typ_ref[...] * N + src_ref[...]
    dg_ref[...] = lax.bitwise_and(dst_ref[...], 7)
    ds_ref[...] = lax.shift_right_logical(dst_ref[...], 3)


def _gidx(src, typ, dst):
    return pl.pallas_call(
        _gidx_body,
        out_shape=[
            jax.ShapeDtypeStruct((NW, NCHUNK, CHUNK), jnp.int32),
            jax.ShapeDtypeStruct((NW, NCHUNK, CHUNK), jnp.int32),
            jax.ShapeDtypeStruct((NW, NCHUNK, CHUNK), jnp.int32),
        ],
    )(src, typ, dst)


K = 2
NGROUP = NCHUNK // K


def _sc_pipe(idx2, dst2, table_h, acc_sh, rows, gsem, ssem, dummy_h):
    def group(g, c):
        @pl.when(g > 0)
        def _():
            for b in range(K):
                pltpu.make_async_copy(dummy_h, rows[b], ssem).wait()

        gd = []
        for b in range(K):
            gd.append(pltpu.async_copy(
                table_h.at[idx2.at[g * K + b]], rows[b], gsem))
        for b in range(K):
            gd[b].wait()
        for b in range(K):
            pltpu.async_copy(rows[b], acc_sh.at[dst2.at[g * K + b]], ssem,
                             add=True)
        return c

    lax.fori_loop(0, NGROUP, group, 0)
    for b in range(K):
        pltpu.make_async_copy(dummy_h, rows[b], ssem).wait()


def _sc_body(gidx_h, dst_h, table_h, zr_h,
             pout_h,
             agg_sh, gidx2, dst2, r0, r1,
             gsem, ssem):
    cid = lax.axis_index("c")
    sid = lax.axis_index("s")
    wid = cid * NS + sid
    row0 = sid * RPT
    pltpu.sync_copy(zr_h, agg_sh.at[pl.ds(row0, RPT)])
    pltpu.sync_copy(gidx_h.at[wid], gidx2)
    pltpu.sync_copy(dst_h.at[wid], dst2)
    plsc.subcore_barrier()
    _sc_pipe(gidx2, dst2, table_h, agg_sh, (r0, r1), gsem, ssem,
             zr_h.at[pl.ds(0, CHUNK)])
    plsc.subcore_barrier()
    pltpu.sync_copy(agg_sh.at[pl.ds(row0, RPT)],
                    pout_h.at[cid, pl.ds(row0, RPT)])


_sc_agg = pl.kernel(
    _sc_body,
    out_type=jax.ShapeDtypeStruct((NC, N_PAD, D), jnp.float32),
    mesh=_MESH,
    scratch_types=[
        pltpu.VMEM_SHARED((N_PAD, D), jnp.float32),
        pltpu.VMEM((NCHUNK, CHUNK), jnp.int32),
        pltpu.VMEM((NCHUNK, CHUNK), jnp.int32),
        pltpu.VMEM((CHUNK, D), jnp.float32),
        pltpu.VMEM((CHUNK, D), jnp.float32),
        pltpu.SemaphoreType.DMA,
        pltpu.SemaphoreType.DMA,
    ],
)


def _sc_deg_body(dg_h, ds_h, onehot_h, zr_h,
                 pdeg_h,
                 deg_sh, dg2, ds2, r0, r1,
                 gsem, ssem):
    cid = lax.axis_index("c")
    sid = lax.axis_index("s")
    wid = cid * NS + sid
    row0 = sid * RPT8
    pltpu.sync_copy(zr_h.at[pl.ds(0, RPT8)], deg_sh.at[pl.ds(row0, RPT8)])
    pltpu.sync_copy(dg_h.at[wid], dg2)
    pltpu.sync_copy(ds_h.at[wid], ds2)
    plsc.subcore_barrier()
    _sc_pipe(dg2, ds2, onehot_h, deg_sh, (r0, r1), gsem, ssem,
             zr_h.at[pl.ds(0, CHUNK)])
    plsc.subcore_barrier()
    pltpu.sync_copy(deg_sh.at[pl.ds(row0, RPT8)],
                    pdeg_h.at[cid, pl.ds(row0, RPT8)])


_sc_deg = pl.kernel(
    _sc_deg_body,
    out_type=jax.ShapeDtypeStruct((NC, ND8, D), jnp.float32),
    mesh=_MESH,
    scratch_types=[
        pltpu.VMEM_SHARED((ND8, D), jnp.float32),
        pltpu.VMEM((NCHUNK, CHUNK), jnp.int32),
        pltpu.VMEM((NCHUNK, CHUNK), jnp.int32),
        pltpu.VMEM((CHUNK, D), jnp.float32),
        pltpu.VMEM((CHUNK, D), jnp.float32),
        pltpu.SemaphoreType.DMA,
        pltpu.SemaphoreType.DMA,
    ],
)


def _degsum_body(pd_ref, d_ref):
    d8 = pd_ref[0] + pd_ref[1]
    ci = lax.broadcasted_iota(jnp.int32, (D, 8), 0)
    ki = lax.broadcasted_iota(jnp.int32, (D, 8), 1)
    sel = (ci == ki * 16).astype(jnp.float32)
    d_ref[...] = lax.dot_general(d8, sel, (((1,), (0,)), ((), ())),
                                 preferred_element_type=jnp.float32)


_BD = 160


def _degsum(pdeg):
    return pl.pallas_call(
        _degsum_body,
        grid=(ND8 // _BD,),
        in_specs=[pl.BlockSpec((NC, _BD, D), lambda i: (0, i, 0))],
        out_specs=pl.BlockSpec((_BD, 8), lambda i: (i, 0)),
        out_shape=jax.ShapeDtypeStruct((ND8, 8), jnp.float32),
    )(pdeg)


def _combine_body(s_ref, p_ref, d_ref, o_ref):
    deg = jnp.maximum(d_ref[...], 1.0)
    agg = p_ref[0] + p_ref[1]
    o_ref[...] = jnp.maximum(s_ref[...] + agg / deg, 0.0)


def _combine(selfout, p, deg):
    return pl.pallas_call(
        _combine_body,
        grid=(N // BN,),
        in_specs=[
            pl.BlockSpec((BN, D), lambda i: (i, 0)),
            pl.BlockSpec((NC, BN, D), lambda i: (0, i, 0)),
            pl.BlockSpec((BN, 1), lambda i: (i, 0)),
        ],
        out_specs=pl.BlockSpec((BN, D), lambda i: (i, 0)),
        out_shape=jax.ShapeDtypeStruct((N, D), jnp.float32),
    )(selfout, p, deg)


def kernel(local_nodes, edge_index_local, edge_type_local, node_emb,
           W_rel0, Wself_w0, Wself_b0, W_rel1, Wself_w1, Wself_b1):
    x = jnp.take(node_emb, local_nodes, axis=0)
    pad = E_PAD - E
    src = jnp.concatenate(
        [edge_index_local[0], jnp.zeros((pad,), jnp.int32)]
    ).reshape(NW, NCHUNK, CHUNK)
    typ = jnp.concatenate(
        [edge_type_local, jnp.zeros((pad,), jnp.int32)]
    ).reshape(NW, NCHUNK, CHUNK)
    dst = jnp.concatenate(
        [edge_index_local[1], jnp.full((pad,), N, jnp.int32)]
    ).reshape(NW, NCHUNK, CHUNK)
    zr = jnp.zeros((RPT, D), jnp.float32)
    onehot = (
        lax.broadcasted_iota(jnp.int32, (8, D), 1)
        == lax.broadcasted_iota(jnp.int32, (8, D), 0) * 16
    ).astype(jnp.float32)

    gidx, dgidx, dsidx = _gidx(src, typ, dst)
    pdeg = _sc_deg(dgidx, dsidx, onehot, zr)
    deg = _degsum(pdeg).reshape(N_PAD, 1)
    t0, s0 = _dense(x, W_rel0, Wself_w0, Wself_b0)
    p0 = _sc_agg(gidx, dst, t0.reshape(R * N, D), zr)
    x1 = _combine(s0, p0, deg)
    t1, s1 = _dense(x1, W_rel1, Wself_w1, Wself_b1)
    p1 = _sc_agg(gidx, dst, t1.reshape(R * N, D), zr)
    x2 = _combine(s1, p1, deg)
    return x2
